# Initial kernel scaffold; baseline (speedup 1.0000x reference)
import jax
import jax.numpy as jnp
from jax.experimental import pallas as pl


def kernel(x, batch_num, gnw, gnb, msc):
    raise NotImplementedError("write your pallas kernel here")


# speedup vs baseline: 6.5445x; 6.5445x over previous
import functools

import jax
import jax.numpy as jnp
from jax import lax
from jax.experimental import pallas as pl
from jax.experimental.pallas import tpu as pltpu
from jax.experimental.pallas import tpu_sc as plsc

NUM_WORKERS = 32
LANES = 16
SEG_CAP = 512
_BITS = (256, 128, 64, 32, 16, 8, 4, 2, 1)


def _extract_i32(vec_ref, idx):
    return vec_ref[pl.ds(idx, LANES)][0]


def _newton_rsqrt(v):
    i = lax.bitcast_convert_type(v, jnp.int32)
    i = jnp.int32(0x5F3759DF) - lax.shift_right_logical(i, 1)
    y = lax.bitcast_convert_type(i, jnp.float32)
    half = v * 0.5
    for _ in range(4):
        y = y * (1.5 - half * y * y)
    return y


def _graphnorm_body(d, x_hbm, sched_hbm, starts_hbm, counts_hbm, gnw_hbm,
                    gnb_hbm, msc_hbm, out_hbm, sched_v, starts_v, counts_v,
                    gnw_v, gnb_v, msc_v, buf):
    ncg = d // LANES
    wid = lax.axis_index("s") * 2 + lax.axis_index("c")

    pltpu.sync_copy(sched_hbm.at[pl.ds(wid * LANES, LANES)], sched_v)
    pltpu.sync_copy(starts_hbm, starts_v)
    pltpu.sync_copy(counts_hbm, counts_v)
    pltpu.sync_copy(gnw_hbm, gnw_v)
    pltpu.sync_copy(gnb_hbm, gnb_v)
    pltpu.sync_copy(msc_hbm, msc_v)

    sv = sched_v[pl.ds(0, LANES)]
    g_lo = sv[0]
    n_segs = sv[1]

    def seg_body(i, carry):
        g = g_lo + i
        o = _extract_i32(starts_v, g)
        c = _extract_i32(counts_v, g)

        for b in _BITS:
            off = (c // (2 * b)) * (2 * b)

            @pl.when((c & b) != 0)
            def _load(off=off, b=b):
                pltpu.sync_copy(x_hbm.at[pl.ds((o + off) * d, b * d)],
                                buf.at[pl.ds(off * d, b * d)])

        def acc_body(j, carry):
            accs, accq = carry
            row = j * d
            new_s = []
            new_q = []
            for cg in range(ncg):
                v = buf[pl.ds(row + cg * LANES, LANES)]
                new_s.append(accs[cg] + v)
                new_q.append(accq[cg] + v * v)
            return tuple(new_s), tuple(new_q)

        zeros = tuple(jnp.zeros((LANES,), jnp.float32) for _ in range(ncg))
        accs, accq = lax.fori_loop(0, c, acc_body, (zeros, zeros))

        rc = 1.0 / lax.broadcast(c.astype(jnp.float32), (LANES,))
        A = []
        Bv = []
        for cg in range(ncg):
            sl = pl.ds(cg * LANES, LANES)
            mean = accs[cg] * rc
            e2 = accq[cg] * rc
            mm = msc_v[sl] * mean
            var = e2 - (2.0 * mean - mm) * mm
            inv = _newton_rsqrt(var + 1e-6)
            a = gnw_v[sl] * inv
            A.append(a)
            Bv.append(gnb_v[sl] - mm * a)

        def norm_body(j, carry):
            row = j * d
            for cg in range(ncg):
                sl = pl.ds(row + cg * LANES, LANES)
                buf[sl] = buf[sl] * A[cg] + Bv[cg]
            return carry

        lax.fori_loop(0, c, norm_body, 0)

        for b in _BITS:
            off = (c // (2 * b)) * (2 * b)

            @pl.when((c & b) != 0)
            def _store(off=off, b=b):
                pltpu.sync_copy(buf.at[pl.ds(off * d, b * d)],
                                out_hbm.at[pl.ds((o + off) * d, b * d)])

        return carry

    lax.fori_loop(0, n_segs, seg_body, 0)


def kernel(x, batch_num, gnw, gnb, msc):
    n, d = x.shape
    b = batch_num.shape[0]
    counts = batch_num.astype(jnp.int32)
    starts = jnp.concatenate(
        [jnp.zeros((1,), jnp.int32), jnp.cumsum(counts)[:-1]])

    targets = (jnp.arange(NUM_WORKERS + 1, dtype=jnp.int32) * n) // NUM_WORKERS
    cuts = jnp.searchsorted(starts, targets, side="left").astype(jnp.int32)
    cuts = cuts.at[0].set(0).at[-1].set(b)
    sched = jnp.zeros((NUM_WORKERS, 16), jnp.int32)
    sched = sched.at[:, 0].set(cuts[:-1]).at[:, 1].set(cuts[1:] - cuts[:-1])

    pad = LANES + ((-b) % LANES)
    starts = jnp.concatenate([starts, jnp.full((pad,), n, jnp.int32)])
    counts = jnp.concatenate([counts, jnp.zeros((pad,), jnp.int32)])
    bp = b + pad

    mesh = plsc.VectorSubcoreMesh(core_axis_name="c", subcore_axis_name="s")
    run = functools.partial(
        pl.kernel,
        mesh=mesh,
        out_type=jax.ShapeDtypeStruct((n * d,), jnp.float32),
        scratch_types=[
            pltpu.VMEM((16,), jnp.int32),
            pltpu.VMEM((bp,), jnp.int32),
            pltpu.VMEM((bp,), jnp.int32),
            pltpu.VMEM((d,), jnp.float32),
            pltpu.VMEM((d,), jnp.float32),
            pltpu.VMEM((d,), jnp.float32),
            pltpu.VMEM((SEG_CAP * d,), jnp.float32),
        ],
    )(functools.partial(_graphnorm_body, d))
    out = run(x.reshape(n * d), sched.reshape(-1), starts, counts, gnw, gnb,
              msc)
    return out.reshape(n, d)


# speedup vs baseline: 15.5254x; 2.3723x over previous
import functools

import jax
import jax.numpy as jnp
from jax import lax
from jax.experimental import pallas as pl
from jax.experimental.pallas import tpu as pltpu
from jax.experimental.pallas import tpu_sc as plsc

NUM_WORKERS = 32
LANES = 16
SEG_CAP = 448
_BITS = (256, 128, 64, 32, 16, 8, 4, 2, 1)


def _extract_i32(vec_ref, idx):
    return vec_ref[pl.ds(idx, LANES)][0]


def _newton_rsqrt(v):
    i = lax.bitcast_convert_type(v, jnp.int32)
    i = jnp.int32(0x5F3759DF) - lax.shift_right_logical(i, 1)
    y = lax.bitcast_convert_type(i, jnp.float32)
    half = v * 0.5
    for _ in range(4):
        y = y * (1.5 - half * y * y)
    return y


def _for_each_chunk(c, fn):
    for b in _BITS:
        off = (c // (2 * b)) * (2 * b)

        @pl.when((c & b) != 0)
        def _go(off=off, b=b):
            fn(off, b)


def _graphnorm_body(d, x_hbm, sched_hbm, starts_hbm, counts_hbm, gnw_hbm,
                    gnb_hbm, msc_hbm, out_hbm, sched_v, starts_v, counts_v,
                    gnw_v, gnb_v, msc_v, buf0, buf1, lsem0, lsem1, ssem0,
                    ssem1):
    ncg = d // LANES
    wid = lax.axis_index("s") * 2 + lax.axis_index("c")

    pltpu.sync_copy(sched_hbm.at[pl.ds(wid * LANES, LANES)], sched_v)
    pltpu.sync_copy(starts_hbm, starts_v)
    pltpu.sync_copy(counts_hbm, counts_v)
    pltpu.sync_copy(gnw_hbm, gnw_v)
    pltpu.sync_copy(gnb_hbm, gnb_v)
    pltpu.sync_copy(msc_hbm, msc_v)

    sv = sched_v[pl.ds(0, LANES)]
    g_lo = sv[0]
    n_segs = sv[1]

    def seg_info(i):
        g = g_lo + i
        return _extract_i32(starts_v, g), _extract_i32(counts_v, g)

    def issue_loads(o, c, buf, sem):
        _for_each_chunk(c, lambda off, b: pltpu.async_copy(
            x_hbm.at[pl.ds((o + off) * d, b * d)],
            buf.at[pl.ds(off * d, b * d)], sem))

    def wait_loads(o, c, buf, sem):
        _for_each_chunk(c, lambda off, b: pltpu.make_async_copy(
            x_hbm.at[pl.ds((o + off) * d, b * d)],
            buf.at[pl.ds(off * d, b * d)], sem).wait())

    def issue_stores(o, c, buf, sem):
        _for_each_chunk(c, lambda off, b: pltpu.async_copy(
            buf.at[pl.ds(off * d, b * d)],
            out_hbm.at[pl.ds((o + off) * d, b * d)], sem))

    def wait_stores(o, c, buf, sem):
        _for_each_chunk(c, lambda off, b: pltpu.make_async_copy(
            buf.at[pl.ds(off * d, b * d)],
            out_hbm.at[pl.ds((o + off) * d, b * d)], sem).wait())

    def compute(c, buf):
        def acc_body(j, carry):
            accs, accq = carry
            row = j * d
            new_s = []
            new_q = []
            for cg in range(ncg):
                v = buf[pl.ds(row + cg * LANES, LANES)]
                new_s.append(accs[cg] + v)
                new_q.append(accq[cg] + v * v)
            return tuple(new_s), tuple(new_q)

        zeros = tuple(jnp.zeros((LANES,), jnp.float32) for _ in range(ncg))
        accs, accq = lax.fori_loop(0, c, acc_body, (zeros, zeros))

        rc = 1.0 / lax.broadcast(c.astype(jnp.float32), (LANES,))
        A = []
        Bv = []
        for cg in range(ncg):
            sl = pl.ds(cg * LANES, LANES)
            mean = accs[cg] * rc
            e2 = accq[cg] * rc
            mm = msc_v[sl] * mean
            var = e2 - (2.0 * mean - mm) * mm
            inv = _newton_rsqrt(var + 1e-6)
            a = gnw_v[sl] * inv
            A.append(a)
            Bv.append(gnb_v[sl] - mm * a)

        def norm_body(j, carry):
            row = j * d
            for cg in range(ncg):
                sl = pl.ds(row + cg * LANES, LANES)
                buf[sl] = buf[sl] * A[cg] + Bv[cg]
            return carry

        lax.fori_loop(0, c, norm_body, 0)

    o0, c0 = seg_info(0)

    @pl.when(n_segs > 0)
    def _prologue():
        issue_loads(o0, c0, buf0, lsem0)

    def body(i, carry):
        def run(cur_buf, cur_l, cur_s, oth_buf, oth_l, oth_s):
            o, c = seg_info(i)

            @pl.when(i + 1 < n_segs)
            def _prefetch():
                on, cn = seg_info(i + 1)

                @pl.when(i >= 1)
                def _drain_prev():
                    op, cp = seg_info(i - 1)
                    wait_stores(op, cp, oth_buf, oth_s)

                issue_loads(on, cn, oth_buf, oth_l)

            wait_loads(o, c, cur_buf, cur_l)
            compute(c, cur_buf)
            issue_stores(o, c, cur_buf, cur_s)

        @pl.when(i % 2 == 0)
        def _even():
            run(buf0, lsem0, ssem0, buf1, lsem1, ssem1)

        @pl.when(i % 2 == 1)
        def _odd():
            run(buf1, lsem1, ssem1, buf0, lsem0, ssem0)

        return carry

    lax.fori_loop(0, n_segs, body, 0)

    @pl.when(n_segs > 1)
    def _drain_m2():
        i = n_segs - 2
        o, c = seg_info(i)

        @pl.when(i % 2 == 0)
        def _e():
            wait_stores(o, c, buf0, ssem0)

        @pl.when(i % 2 == 1)
        def _o():
            wait_stores(o, c, buf1, ssem1)

    @pl.when(n_segs > 0)
    def _drain_m1():
        i = n_segs - 1
        o, c = seg_info(i)

        @pl.when(i % 2 == 0)
        def _e():
            wait_stores(o, c, buf0, ssem0)

        @pl.when(i % 2 == 1)
        def _o():
            wait_stores(o, c, buf1, ssem1)


def kernel(x, batch_num, gnw, gnb, msc):
    n, d = x.shape
    b = batch_num.shape[0]
    counts = batch_num.astype(jnp.int32)
    starts = jnp.concatenate(
        [jnp.zeros((1,), jnp.int32), jnp.cumsum(counts)[:-1]])

    targets = (jnp.arange(NUM_WORKERS + 1, dtype=jnp.int32) * n) // NUM_WORKERS
    cuts = jnp.searchsorted(starts, targets, side="left").astype(jnp.int32)
    cuts = cuts.at[0].set(0).at[-1].set(b)
    sched = jnp.zeros((NUM_WORKERS, 16), jnp.int32)
    sched = sched.at[:, 0].set(cuts[:-1]).at[:, 1].set(cuts[1:] - cuts[:-1])

    pad = LANES + ((-b) % LANES)
    starts = jnp.concatenate([starts, jnp.full((pad,), n, jnp.int32)])
    counts = jnp.concatenate([counts, jnp.zeros((pad,), jnp.int32)])
    bp = b + pad

    mesh = plsc.VectorSubcoreMesh(core_axis_name="c", subcore_axis_name="s")
    run = functools.partial(
        pl.kernel,
        mesh=mesh,
        out_type=jax.ShapeDtypeStruct((n * d,), jnp.float32),
        scratch_types=[
            pltpu.VMEM((16,), jnp.int32),
            pltpu.VMEM((bp,), jnp.int32),
            pltpu.VMEM((bp,), jnp.int32),
            pltpu.VMEM((d,), jnp.float32),
            pltpu.VMEM((d,), jnp.float32),
            pltpu.VMEM((d,), jnp.float32),
            pltpu.VMEM((SEG_CAP * d,), jnp.float32),
            pltpu.VMEM((SEG_CAP * d,), jnp.float32),
            pltpu.SemaphoreType.DMA,
            pltpu.SemaphoreType.DMA,
            pltpu.SemaphoreType.DMA,
            pltpu.SemaphoreType.DMA,
        ],
    )(functools.partial(_graphnorm_body, d))
    out = run(x.reshape(n * d), sched.reshape(-1), starts, counts, gnw, gnb,
              msc)
    return out.reshape(n, d)
